# initial kernel scaffold (unmeasured)
import jax
import jax.numpy as jnp
from jax import lax
from jax.experimental import pallas as pl
from jax.experimental.pallas import tpu as pltpu

N_DEV = 8


def kernel(x, Wp):
    B, Hs, W, C = x.shape
    Cout = Wp.shape[1]
    n_global = Hs * N_DEV * W
    eps = 1e-5

    def body(x_ref, wp_ref, out_ref, mine_ref, comm_ref, send_sems, recv_sems):
        my = lax.axis_index("i")

        xv = x_ref[...]
        xr = xv.reshape(B, Hs * W, C)
        s1 = jnp.sum(xr, axis=1)
        s2 = jnp.sum(xr * xr, axis=1)
        mine_ref[0, 0:B, :] = s1
        mine_ref[0, B:2 * B, :] = s2
        comm_ref[pl.ds(my, 1)] = mine_ref[...]

        sends = []
        for off in range(1, N_DEV):
            tgt = lax.rem(my + off, N_DEV)
            rdma = pltpu.make_async_remote_copy(
                src_ref=mine_ref,
                dst_ref=comm_ref.at[pl.ds(my, 1)],
                send_sem=send_sems.at[off],
                recv_sem=recv_sems.at[my],
                device_id=(tgt,),
                device_id_type=pl.DeviceIdType.MESH,
            )
            rdma.start()
            sends.append(rdma)

        for s in range(N_DEV):
            @pl.when(s != my)
            def _():
                recv = pltpu.make_async_remote_copy(
                    src_ref=mine_ref,
                    dst_ref=comm_ref.at[pl.ds(s, 1)],
                    send_sem=send_sems.at[0],
                    recv_sem=recv_sems.at[s],
                    device_id=(my,),
                    device_id_type=pl.DeviceIdType.MESH,
                )
                recv.wait_recv()

        tot = jnp.sum(comm_ref[...], axis=0)
        tot = tot.reshape(2 * B, C)
        s1t = tot[0:B, :]
        s2t = tot[B:2 * B, :]
        mean = s1t / n_global
        var = s2t / n_global - mean * mean
        rstd = lax.rsqrt(var + eps)

        h = (xr - mean[:, None, :]) * rstd[:, None, :]
        a = h * (1.0 / (1.0 + jnp.exp(-h)))
        a2 = a.reshape(B * Hs * W, C)
        res = jnp.dot(a2, wp_ref[...], preferred_element_type=jnp.float32)
        out_ref[...] = res.reshape(B, Hs, W, Cout)

        for rdma in sends:
            rdma.wait_send()

    return pl.pallas_call(
        body,
        out_shape=jax.ShapeDtypeStruct((B, Hs, W, Cout), jnp.float32),
        in_specs=[
            pl.BlockSpec(memory_space=pltpu.VMEM),
            pl.BlockSpec(memory_space=pltpu.VMEM),
        ],
        out_specs=pl.BlockSpec(memory_space=pltpu.VMEM),
        scratch_shapes=[
            pltpu.VMEM((1, 2 * B, C), jnp.float32),
            pltpu.VMEM((N_DEV, 2 * B, C), jnp.float32),
            pltpu.SemaphoreType.DMA((N_DEV,)),
            pltpu.SemaphoreType.DMA((N_DEV,)),
        ],
        compiler_params=pltpu.CompilerParams(collective_id=0),
    )(x, Wp)


# baseline (device time: 18126 ns/iter reference)
import jax
import jax.numpy as jnp
from jax import lax
from jax.experimental import pallas as pl
from jax.experimental.pallas import tpu as pltpu

N_DEV = 8


def kernel(x, Wp):
    B, Hs, W, C = x.shape
    Cout = Wp.shape[1]
    n_global = Hs * N_DEV * W
    eps = 1e-5

    def body(x_ref, wp_ref, out_ref, mine_ref, comm_ref, send_sems, recv_sems):
        my = lax.axis_index("i")

        xv = x_ref[...]
        xr = xv.reshape(B, Hs * W, C)
        s1 = jnp.sum(xr, axis=1)
        s2 = jnp.sum(xr * xr, axis=1)
        mine_ref[0, 0:B, :] = s1
        mine_ref[0, B:2 * B, :] = s2
        comm_ref[pl.ds(my, 1)] = mine_ref[...]

        sends = []
        for off in range(1, N_DEV):
            tgt = lax.rem(my + off, N_DEV)
            rdma = pltpu.make_async_remote_copy(
                src_ref=mine_ref,
                dst_ref=comm_ref.at[pl.ds(my, 1)],
                send_sem=send_sems.at[off],
                recv_sem=recv_sems.at[my],
                device_id=(tgt,),
                device_id_type=pl.DeviceIdType.MESH,
            )
            rdma.start()
            sends.append(rdma)

        for s in range(N_DEV):
            @pl.when(s != my)
            def _():
                recv = pltpu.make_async_remote_copy(
                    src_ref=mine_ref,
                    dst_ref=comm_ref.at[pl.ds(s, 1)],
                    send_sem=send_sems.at[0],
                    recv_sem=recv_sems.at[s],
                    device_id=(my,),
                    device_id_type=pl.DeviceIdType.MESH,
                )
                recv.wait_recv()

        tot = jnp.sum(comm_ref[...], axis=0)
        tot = tot.reshape(2 * B, C)
        s1t = tot[0:B, :]
        s2t = tot[B:2 * B, :]
        mean = s1t / n_global
        var = s2t / n_global - mean * mean
        rstd = lax.rsqrt(var + eps)

        h = (xr - mean[:, None, :]) * rstd[:, None, :]
        a = h * (1.0 / (1.0 + jnp.exp(-h)))
        a2 = a.reshape(B * Hs * W, C)
        res = jnp.dot(a2, wp_ref[...], preferred_element_type=jnp.float32)
        out_ref[...] = res.reshape(B, Hs, W, Cout)

        for rdma in sends:
            rdma.wait_send()

    return pl.pallas_call(
        body,
        out_shape=jax.ShapeDtypeStruct((B, Hs, W, Cout), jnp.float32),
        in_specs=[
            pl.BlockSpec(memory_space=pltpu.VMEM),
            pl.BlockSpec(memory_space=pltpu.VMEM),
        ],
        out_specs=pl.BlockSpec(memory_space=pltpu.VMEM),
        scratch_shapes=[
            pltpu.VMEM((1, 2 * B, C), jnp.float32),
            pltpu.VMEM((N_DEV, 2 * B, C), jnp.float32),
            pltpu.SemaphoreType.DMA((N_DEV,)),
            pltpu.SemaphoreType.DMA((N_DEV,)),
        ],
    )(x, Wp)


# device time: 13360 ns/iter; 1.3567x vs baseline; 1.3567x over previous
import jax
import jax.numpy as jnp
from jax import lax
from jax.experimental import pallas as pl
from jax.experimental.pallas import tpu as pltpu

N_DEV = 8


def kernel(x, Wp):
    B, Hs, W, C = x.shape
    Cout = Wp.shape[1]
    n_global = Hs * N_DEV * W
    eps = 1e-5

    def body(x_ref, wp_ref, out_ref, comm_ref, send_sems, recv_sems):
        my = lax.axis_index("i")

        barrier = pltpu.get_barrier_semaphore()
        for off in range(1, N_DEV):
            pl.semaphore_signal(
                barrier, inc=1,
                device_id=(lax.rem(my + off, N_DEV),),
                device_id_type=pl.DeviceIdType.MESH,
            )

        xv = x_ref[...]
        xr = xv.reshape(B, Hs * W, C)
        s1 = jnp.sum(xr, axis=1)
        s2 = jnp.sum(xr * xr, axis=1)
        mine = pl.ds(my, 1)
        comm_ref[mine, 0:B, :] = s1[None]
        comm_ref[mine, B:2 * B, :] = s2[None]

        pl.semaphore_wait(barrier, N_DEV - 1)

        sends = []
        for off in range(1, N_DEV):
            tgt = lax.rem(my + off, N_DEV)
            rdma = pltpu.make_async_remote_copy(
                src_ref=comm_ref.at[mine],
                dst_ref=comm_ref.at[mine],
                send_sem=send_sems.at[off],
                recv_sem=recv_sems.at[my],
                device_id=(tgt,),
                device_id_type=pl.DeviceIdType.MESH,
            )
            rdma.start()
            sends.append(rdma)

        for s in range(N_DEV):
            @pl.when(s != my)
            def _():
                recv = pltpu.make_async_remote_copy(
                    src_ref=comm_ref.at[pl.ds(s, 1)],
                    dst_ref=comm_ref.at[pl.ds(s, 1)],
                    send_sem=send_sems.at[0],
                    recv_sem=recv_sems.at[s],
                    device_id=(my,),
                    device_id_type=pl.DeviceIdType.MESH,
                )
                recv.wait_recv()

        tot = jnp.sum(comm_ref[...], axis=0)
        s1t = tot[0:B, :]
        s2t = tot[B:2 * B, :]
        mean = s1t / n_global
        var = s2t / n_global - mean * mean
        rstd = lax.rsqrt(var + eps)

        h = (xr - mean[:, None, :]) * rstd[:, None, :]
        a = h * (1.0 / (1.0 + jnp.exp(-h)))
        a2 = a.reshape(B * Hs * W, C)
        res = jnp.dot(a2, wp_ref[...], preferred_element_type=jnp.float32)
        out_ref[...] = res.reshape(B, Hs, W, Cout)

        for rdma in sends:
            rdma.wait_send()

    return pl.pallas_call(
        body,
        out_shape=jax.ShapeDtypeStruct((B, Hs, W, Cout), jnp.float32),
        in_specs=[
            pl.BlockSpec(memory_space=pltpu.VMEM),
            pl.BlockSpec(memory_space=pltpu.VMEM),
        ],
        out_specs=pl.BlockSpec(memory_space=pltpu.VMEM),
        scratch_shapes=[
            pltpu.VMEM((N_DEV, 2 * B, C), jnp.float32),
            pltpu.SemaphoreType.DMA((N_DEV,)),
            pltpu.SemaphoreType.DMA((N_DEV,)),
        ],
        compiler_params=pltpu.CompilerParams(collective_id=0),
    )(x, Wp)
